# baseline (speedup 1.0000x reference)
import functools

import jax
import jax.numpy as jnp
from jax import lax
from jax.experimental import pallas as pl
from jax.experimental.pallas import tpu as pltpu
from jax.experimental.pallas import tpu_sc as plsc

BATCH = 16384
NUM_FIELDS = 26
FIELD_SIZE = 100000
L = 16
NW = 32
BPW = BATCH // NW
WORDS = BPW * NUM_FIELDS
IDX_ROWS = WORDS // 128

_mesh = plsc.VectorSubcoreMesh(core_axis_name="c", subcore_axis_name="s")


@functools.partial(
    pl.kernel,
    out_type=jax.ShapeDtypeStruct((BATCH,), jnp.float32),
    mesh=_mesh,
    compiler_params=pltpu.CompilerParams(needs_layout_passes=False, disable_semaphore_checks=True, skip_device_barrier=True),
    scratch_types=[
        pltpu.VMEM((NUM_FIELDS, BPW), jnp.int32),
        pltpu.VMEM((IDX_ROWS, 128), jnp.int32),
        pltpu.VMEM((IDX_ROWS, 128), jnp.float32),
        pltpu.VMEM((BPW,), jnp.float32),
        pltpu.VMEM((L,), jnp.float32),
        pltpu.SemaphoreType.DMA,
    ],
)
def _sc_kernel(x_hbm, table_hbm, bias_hbm, out_hbm,
               x_v, idx_v, rows_v, out_v, bias_v, sem):
    wid = lax.axis_index("s") * 2 + lax.axis_index("c")
    base = wid * BPW

    pltpu.sync_copy(x_hbm.at[:, pl.ds(base, BPW)], x_v)
    pltpu.sync_copy(bias_hbm, bias_v)

    for r in range(IDX_ROWS):
        f, c = divmod(r, BPW // 128)
        for k in range(8):
            col = c * 128 + k * L
            idx_v[r, pl.ds(k * L, L)] = x_v[f, pl.ds(col, L)] \
                + f * FIELD_SIZE
        pltpu.make_async_copy(
            table_hbm.at[idx_v.at[r]], rows_v.at[r], sem).start()

    for r in range(IDX_ROWS):
        pltpu.make_async_copy(
            table_hbm.at[idx_v.at[r]], rows_v.at[r], sem).wait()

    bias_vec = bias_v[...]

    for j in range(BPW // L):
        c = j // 8
        col = (j % 8) * L
        acc = bias_vec
        for f in range(NUM_FIELDS):
            acc = acc + rows_v[f * (BPW // 128) + c, pl.ds(col, L)]
        out_v[pl.ds(j * L, L)] = acc

    pltpu.sync_copy(out_v, out_hbm.at[pl.ds(base, BPW)])


def kernel(x, fc_weight, bias):
    out = _sc_kernel(x.T, fc_weight.reshape(-1),
                     jnp.broadcast_to(bias, (L,)))
    return out.reshape(BATCH, 1)


# speedup vs baseline: 1.0002x; 1.0002x over previous
import functools

import jax
import jax.numpy as jnp
from jax import lax
from jax.experimental import pallas as pl
from jax.experimental.pallas import tpu as pltpu
from jax.experimental.pallas import tpu_sc as plsc

BATCH = 16384
NUM_FIELDS = 26
FIELD_SIZE = 100000
L = 16
NW = 32
BPW = BATCH // NW
WORDS = BPW * NUM_FIELDS
IDX_ROWS = WORDS // 128

_mesh = plsc.VectorSubcoreMesh(core_axis_name="c", subcore_axis_name="s")


@functools.partial(
    pl.kernel,
    out_type=jax.ShapeDtypeStruct((BATCH,), jnp.float32),
    mesh=_mesh,
    compiler_params=pltpu.CompilerParams(needs_layout_passes=False),
    scratch_types=[
        pltpu.VMEM((NUM_FIELDS, BPW), jnp.int32),
        pltpu.VMEM((IDX_ROWS, 128), jnp.int32),
        pltpu.VMEM((IDX_ROWS, 128), jnp.float32),
        pltpu.VMEM((BPW,), jnp.float32),
        pltpu.VMEM((L,), jnp.float32),
        pltpu.SemaphoreType.DMA,
    ],
)
def _sc_kernel(x_hbm, table_hbm, bias_hbm, out_hbm,
               x_v, idx_v, rows_v, out_v, bias_v, sem):
    wid = lax.axis_index("s") * 2 + lax.axis_index("c")
    base = wid * BPW

    pltpu.sync_copy(x_hbm.at[:, pl.ds(base, BPW)], x_v)
    pltpu.sync_copy(bias_hbm, bias_v)

    for r in range(IDX_ROWS):
        f, c = divmod(r, BPW // 128)
        for k in range(8):
            col = c * 128 + k * L
            idx_v[r, pl.ds(k * L, L)] = x_v[f, pl.ds(col, L)] \
                + f * FIELD_SIZE
        pltpu.make_async_copy(
            table_hbm.at[idx_v.at[r]], rows_v.at[r], sem).start()

    for r in range(IDX_ROWS):
        pltpu.make_async_copy(
            table_hbm.at[idx_v.at[r]], rows_v.at[r], sem).wait()

    bias_vec = bias_v[...]

    for j in range(BPW // L):
        c = j // 8
        col = (j % 8) * L
        acc = bias_vec
        for f in range(NUM_FIELDS):
            acc = acc + rows_v[f * (BPW // 128) + c, pl.ds(col, L)]
        out_v[pl.ds(j * L, L)] = acc

    pltpu.sync_copy(out_v, out_hbm.at[pl.ds(base, BPW)])


def kernel(x, fc_weight, bias):
    out = _sc_kernel(x.T, fc_weight.reshape(-1),
                     jnp.broadcast_to(bias, (L,)))
    return out.reshape(BATCH, 1)


# speedup vs baseline: 1.0143x; 1.0141x over previous
import functools

import jax
import jax.numpy as jnp
from jax import lax
from jax.experimental import pallas as pl
from jax.experimental.pallas import tpu as pltpu
from jax.experimental.pallas import tpu_sc as plsc

BATCH = 16384
NUM_FIELDS = 26
FIELD_SIZE = 100000
L = 16
NW = 32
BPW = BATCH // NW
WORDS = BPW * NUM_FIELDS
IDX_ROWS = WORDS // 128

_mesh = plsc.VectorSubcoreMesh(core_axis_name="c", subcore_axis_name="s")


@functools.partial(
    pl.kernel,
    out_type=jax.ShapeDtypeStruct((BATCH,), jnp.float32),
    mesh=_mesh,
    compiler_params=pltpu.CompilerParams(needs_layout_passes=False),
    scratch_types=[
        pltpu.VMEM((NUM_FIELDS, BPW), jnp.int32),
        pltpu.VMEM((IDX_ROWS, 128), jnp.int32),
        pltpu.VMEM((IDX_ROWS, 128), jnp.float32),
        pltpu.VMEM((BPW,), jnp.float32),
        pltpu.VMEM((L,), jnp.float32),
        pltpu.SemaphoreType.DMA,
        pltpu.SemaphoreType.DMA,
        pltpu.SemaphoreType.DMA,
        pltpu.SemaphoreType.DMA,
    ],
)
def _sc_kernel(x_hbm, table_hbm, bias_hbm, out_hbm,
               x_v, idx_v, rows_v, out_v, bias_v,
               sem0, sem1, sem2, sem3):
    wid = lax.axis_index("s") * 2 + lax.axis_index("c")
    base = wid * BPW
    sems = (sem0, sem1, sem2, sem3)
    nchunk = BPW // 128

    pltpu.sync_copy(x_hbm.at[:, pl.ds(base, BPW)], x_v)
    pltpu.sync_copy(bias_hbm, bias_v)

    for c in range(nchunk):
        for f in range(NUM_FIELDS):
            r = f * nchunk + c
            for k in range(8):
                col = c * 128 + k * L
                idx_v[r, pl.ds(k * L, L)] = x_v[f, pl.ds(col, L)] \
                    + f * FIELD_SIZE
            pltpu.make_async_copy(
                table_hbm.at[idx_v.at[r]], rows_v.at[r], sems[c]).start()

    bias_vec = bias_v[...]

    for c in range(nchunk):
        for f in range(NUM_FIELDS):
            r = f * nchunk + c
            pltpu.make_async_copy(
                table_hbm.at[idx_v.at[r]], rows_v.at[r], sems[c]).wait()
        for kk in range(8):
            j = c * 8 + kk
            acc = bias_vec
            for f in range(NUM_FIELDS):
                acc = acc + rows_v[f * nchunk + c, pl.ds(kk * L, L)]
            out_v[pl.ds(j * L, L)] = acc
        pltpu.sync_copy(out_v.at[pl.ds(c * 128, 128)],
                        out_hbm.at[pl.ds(base + c * 128, 128)])


def kernel(x, fc_weight, bias):
    out = _sc_kernel(x.T, fc_weight.reshape(-1),
                     jnp.broadcast_to(bias, (L,)))
    return out.reshape(BATCH, 1)
